# baseline (speedup 1.0000x reference)
import jax
import jax.numpy as jnp
from jax.experimental import pallas as pl

P = 64
KTOP = 16
GRID_WD = 8
G = 32


def _body(x_ref, w1_ref, b1_ref, w2_ref, b2_ref, wo_ref, bo_ref,
          ow1_ref, ob1_ref, ow2_ref, ob2_ref, owo_ref, obo_ref,
          out_ref):
    R = G * P
    f32 = jnp.float32
    bf16 = jnp.bfloat16

    s_blocks = []
    for g in range(G):
        xg = x_ref[g]
        s_blocks.append(jax.lax.dot_general(
            xg, xg, (((1,), (1,)), ((), ())), preferred_element_type=f32))
    S = jnp.concatenate(s_blocks, axis=1)

    qv = jax.lax.broadcasted_iota(jnp.int32, (1, P), 1).astype(bf16)
    neg = f32(-jnp.inf)
    vals, idxs = [], []
    for _ in range(KTOP):
        m = jnp.max(S, axis=0, keepdims=True)
        ismax = S == m
        a = jax.lax.dot_general(
            qv, ismax.astype(bf16), (((1,), (0,)), ((), ())),
            preferred_element_type=f32)
        vals.append(m)
        idxs.append(a)
        S = jnp.where(ismax, neg, S)
    V = jnp.concatenate(vals, axis=0)
    I = jnp.concatenate(idxs, axis=0)

    lane = jax.lax.broadcasted_iota(jnp.int32, (KTOP, R), 1)
    p = (lane % P).astype(f32)
    ph = jnp.floor(p * 0.125)
    pw = p - 8.0 * ph
    ih = jnp.floor(I * 0.125)
    iw = I - 8.0 * ih
    inv = f32(1.0 / (GRID_WD - 1))
    dh = (ph - ih) * inv
    dw = (pw - iw) * inv

    rel = jnp.stack([dh, dw], axis=1).reshape(2 * KTOP, R)
    featT = jnp.concatenate([V, rel], axis=0).astype(bf16)

    tr = lambda ref: jnp.transpose(ref[...].astype(bf16), (1, 0))
    hT = jnp.maximum(jnp.dot(tr(w1_ref), featT,
                             preferred_element_type=f32) + b1_ref[...], 0.0)
    hT = jnp.maximum(jnp.dot(tr(w2_ref), hT.astype(bf16),
                             preferred_element_type=f32) + b2_ref[...], 0.0)
    sfT = jnp.dot(tr(wo_ref), hT.astype(bf16),
                  preferred_element_type=f32) + bo_ref[...]

    sf = jnp.transpose(sfT.astype(bf16), (1, 0))
    x2d = x_ref[...].reshape(R, x_ref.shape[2]).astype(bf16)
    y = jnp.dot(jnp.concatenate([x2d, sf], axis=1),
                ow1_ref[...].astype(bf16), preferred_element_type=f32)
    y = jnp.maximum(y + ob1_ref[...], 0.0).astype(bf16)
    y = jnp.maximum(jnp.dot(y, ow2_ref[...].astype(bf16),
                            preferred_element_type=f32)
                    + ob2_ref[...], 0.0).astype(bf16)
    out_ref[...] = (jnp.dot(y, owo_ref[...].astype(bf16),
                            preferred_element_type=f32)
                    + obo_ref[...])


def _forward(inputs, sf_W1, sf_b1, sf_W2, sf_b2, sf_Wo, sf_bo,
             out_W1, out_b1, out_W2, out_b2, out_Wo, out_bo,
             interpret=False):
    B, Wn, P_, C = inputs.shape
    NW = B * Wn
    x = inputs.reshape(NW, P_, C)

    col = lambda b: b.reshape(-1, 1)
    row = lambda b: b.reshape(1, -1)
    weights = (sf_W1, col(sf_b1), sf_W2, col(sf_b2), sf_Wo, col(sf_bo),
               out_W1, row(out_b1), out_W2, row(out_b2),
               out_Wo, row(out_bo))

    wspecs = [pl.BlockSpec(w.shape, lambda i: (0, 0)) for w in weights]
    out = pl.pallas_call(
        _body,
        grid=(NW // G,),
        in_specs=[pl.BlockSpec((G, P_, C), lambda i: (i, 0, 0))] + wspecs,
        out_specs=pl.BlockSpec((G * P_, 256), lambda i: (i, 0)),
        out_shape=jax.ShapeDtypeStruct((NW * P_, 256), jnp.float32),
        interpret=interpret,
    )(x, *weights)
    return out.reshape(B, Wn, P_, 256)


def kernel(inputs, sf_W1, sf_b1, sf_W2, sf_b2, sf_Wo, sf_bo,
           out_W1, out_b1, out_W2, out_b2, out_Wo, out_bo):
    return _forward(inputs, sf_W1, sf_b1, sf_W2, sf_b2, sf_Wo, sf_bo,
                    out_W1, out_b1, out_W2, out_b2, out_Wo, out_bo)


# speedup vs baseline: 1.0652x; 1.0652x over previous
import jax
import jax.numpy as jnp
from jax.experimental import pallas as pl

P = 64
KTOP = 16
GRID_WD = 8
G = 64


def _body(x_ref, w1_ref, b1_ref, w2_ref, b2_ref, wo_ref, bo_ref,
          ow1_ref, ob1_ref, ow2_ref, ob2_ref, owo_ref, obo_ref,
          out_ref):
    R = G * P
    f32 = jnp.float32
    bf16 = jnp.bfloat16

    s_blocks = []
    for g in range(G):
        xg = x_ref[g]
        s_blocks.append(jax.lax.dot_general(
            xg, xg, (((1,), (1,)), ((), ())), preferred_element_type=f32))
    S = jnp.concatenate(s_blocks, axis=1)

    neg = f32(-jnp.inf)
    vals, masks = [], []
    for _ in range(KTOP):
        m = jnp.max(S, axis=0, keepdims=True)
        ismax = S == m
        vals.append(m)
        masks.append(ismax.astype(bf16))
        S = jnp.where(ismax, neg, S)
    V = jnp.concatenate(vals, axis=0)

    kk = jax.lax.broadcasted_iota(jnp.int32, (KTOP, KTOP * P), 0)
    cc = jax.lax.broadcasted_iota(jnp.int32, (KTOP, KTOP * P), 1)
    qsel = jnp.where(cc // P == kk, cc % P, 0).astype(bf16)
    I = jnp.dot(qsel, jnp.concatenate(masks, axis=0),
                preferred_element_type=f32)

    lane = jax.lax.broadcasted_iota(jnp.int32, (KTOP, R), 1)
    p = (lane % P).astype(f32)
    ph = jnp.floor(p * 0.125)
    pw = p - 8.0 * ph
    ih = jnp.floor(I * 0.125)
    iw = I - 8.0 * ih
    inv = f32(1.0 / (GRID_WD - 1))
    dh = (ph - ih) * inv
    dw = (pw - iw) * inv

    rel = jnp.stack([dh, dw], axis=1).reshape(2 * KTOP, R)
    featT = jnp.concatenate([V, rel], axis=0).astype(bf16)

    tr = lambda ref: jnp.transpose(ref[...].astype(bf16), (1, 0))
    hT = jnp.maximum(jnp.dot(tr(w1_ref), featT,
                             preferred_element_type=f32) + b1_ref[...], 0.0)
    hT = jnp.maximum(jnp.dot(tr(w2_ref), hT.astype(bf16),
                             preferred_element_type=f32) + b2_ref[...], 0.0)
    sfT = jnp.dot(tr(wo_ref), hT.astype(bf16),
                  preferred_element_type=f32) + bo_ref[...]

    sf = jnp.transpose(sfT.astype(bf16), (1, 0))
    x2d = x_ref[...].reshape(R, x_ref.shape[2]).astype(bf16)
    y = jnp.dot(jnp.concatenate([x2d, sf], axis=1),
                ow1_ref[...].astype(bf16), preferred_element_type=f32)
    y = jnp.maximum(y + ob1_ref[...], 0.0).astype(bf16)
    y = jnp.maximum(jnp.dot(y, ow2_ref[...].astype(bf16),
                            preferred_element_type=f32)
                    + ob2_ref[...], 0.0).astype(bf16)
    out_ref[...] = (jnp.dot(y, owo_ref[...].astype(bf16),
                            preferred_element_type=f32)
                    + obo_ref[...])


def _forward(inputs, sf_W1, sf_b1, sf_W2, sf_b2, sf_Wo, sf_bo,
             out_W1, out_b1, out_W2, out_b2, out_Wo, out_bo,
             interpret=False):
    B, Wn, P_, C = inputs.shape
    NW = B * Wn
    x = inputs.reshape(NW, P_, C)

    col = lambda b: b.reshape(-1, 1)
    row = lambda b: b.reshape(1, -1)
    weights = (sf_W1, col(sf_b1), sf_W2, col(sf_b2), sf_Wo, col(sf_bo),
               out_W1, row(out_b1), out_W2, row(out_b2),
               out_Wo, row(out_bo))

    wspecs = [pl.BlockSpec(w.shape, lambda i: (0, 0)) for w in weights]
    out = pl.pallas_call(
        _body,
        grid=(NW // G,),
        in_specs=[pl.BlockSpec((G, P_, C), lambda i: (i, 0, 0))] + wspecs,
        out_specs=pl.BlockSpec((G * P_, 256), lambda i: (i, 0)),
        out_shape=jax.ShapeDtypeStruct((NW * P_, 256), jnp.float32),
        interpret=interpret,
    )(x, *weights)
    return out.reshape(B, Wn, P_, 256)


def kernel(inputs, sf_W1, sf_b1, sf_W2, sf_b2, sf_Wo, sf_bo,
           out_W1, out_b1, out_W2, out_b2, out_Wo, out_bo):
    return _forward(inputs, sf_W1, sf_b1, sf_W2, sf_b2, sf_Wo, sf_bo,
                    out_W1, out_b1, out_W2, out_b2, out_Wo, out_bo)
